# baseline (speedup 1.0000x reference)
import jax
import jax.numpy as jnp
from jax.experimental import pallas as pl
from jax.experimental.pallas import tpu as pltpu

EMB = 64
HIST = 200
ROW = HIST * EMB
TB = 256
NSEM = 8


def _stream_kernel(p_ref, o_ref, scratch, sems):
    scratch[...] = jnp.broadcast_to(p_ref[...], scratch.shape)
    nchunks = o_ref.shape[0] // TB

    def copy(i):
        return pltpu.make_async_copy(
            scratch, o_ref.at[pl.ds(i * TB, TB), :], sems.at[i % NSEM]
        )

    for i in range(nchunks):
        if i >= NSEM:
            copy(i - NSEM).wait()
        copy(i).start()
    for i in range(max(0, nchunks - NSEM), nchunks):
        copy(i).wait()


def kernel(sequence, param):
    batch = sequence.shape[0]
    row = jnp.tile(param, HIST).reshape(1, ROW)
    out = pl.pallas_call(
        _stream_kernel,
        in_specs=[pl.BlockSpec(memory_space=pltpu.MemorySpace.VMEM)],
        out_specs=pl.BlockSpec(memory_space=pl.ANY),
        out_shape=jax.ShapeDtypeStruct((batch, ROW), jnp.float32),
        scratch_shapes=[
            pltpu.VMEM((TB, ROW), jnp.float32),
            pltpu.SemaphoreType.DMA((NSEM,)),
        ],
    )(row)
    return out.reshape(batch, HIST, EMB)
